# baseline (speedup 1.0000x reference)
import functools
import math

import jax
import jax.numpy as jnp
from jax import lax
from jax.experimental import pallas as pl
from jax.experimental.pallas import tpu as pltpu
from jax.experimental.pallas import tpu_sc as plsc

N = 10000
E = 320000
D = 128
H = 128
PE = 32
OUT = 10

NC = 2
NS = 16
NW = NC * NS
CHUNK = 128
ROWS_T = 640
NPAD = NS * ROWS_T
EC = 125
EC8 = 128
ECH_T = (E // NW) // EC

R = 2000
G = N // R
EPS = 1e-5
BN_SCALE = 1.0 / math.sqrt(1.0 + EPS)

_f32 = jnp.float32
_mesh = plsc.VectorSubcoreMesh(core_axis_name="c", subcore_axis_name="s",
                               num_cores=NC, num_subcores=NS)


@functools.partial(
    pl.kernel,
    out_type=[jax.ShapeDtypeStruct((NPAD,), _f32),
              jax.ShapeDtypeStruct((NPAD,), _f32)],
    mesh=_mesh,
    scratch_types=[
        pltpu.VMEM((EC,), jnp.int32),
        pltpu.VMEM((EC,), jnp.int32),
        pltpu.VMEM((EC8,), _f32),
        pltpu.VMEM((CHUNK,), _f32),
        pltpu.VMEM_SHARED((NPAD,), _f32),
        pltpu.SemaphoreType.DMA,
        pltpu.SemaphoreType.DMA,
    ],
)
def _sc_degree(dst_hbm, out0_hbm, out1_hbm, dia, dib, ones, zf, acc,
               dsema, dsemb):
    c = lax.axis_index("c")
    s = lax.axis_index("s")
    start = s * ROWS_T
    wid = s * NC + c
    base = wid * ECH_T
    for k in range(EC8 // 16):
        ones[pl.ds(k * 16, 16)] = jnp.ones((16,), _f32)
    for k in range(CHUNK // 16):
        zf[pl.ds(k * 16, 16)] = jnp.zeros((16,), _f32)
    zds = [pltpu.async_copy(zf, acc.at[pl.ds(start + q * CHUNK, CHUNK)],
                            dsema) for q in range(ROWS_T // CHUNK)]
    for zd in zds:
        zd.wait()
    plsc.subcore_barrier()

    pltpu.sync_copy(dst_hbm.at[base], dia)

    def pair(j, carry):
        c0 = 2 * j
        ddb = pltpu.async_copy(dst_hbm.at[base + c0 + 1], dib, dsemb)
        pltpu.sync_copy(ones.at[pl.ds(0, EC)], acc.at[dia], add=True)
        ddb.wait()

        @pl.when(j < ECH_T // 2 - 1)
        def _():
            dda = pltpu.async_copy(dst_hbm.at[base + c0 + 2], dia, dsema)
            pltpu.sync_copy(ones.at[pl.ds(0, EC)], acc.at[dib], add=True)
            dda.wait()

        @pl.when(j == ECH_T // 2 - 1)
        def _():
            pltpu.sync_copy(ones.at[pl.ds(0, EC)], acc.at[dib], add=True)

        return carry

    lax.fori_loop(0, ECH_T // 2, pair, 0)
    plsc.subcore_barrier()
    for cc, out_hbm in ((0, out0_hbm), (1, out1_hbm)):
        @pl.when(c == cc)
        def _():
            for q in range(ROWS_T // CHUNK):
                pltpu.sync_copy(acc.at[pl.ds(start + q * CHUNK, CHUNK)], zf)
                pltpu.sync_copy(zf, out_hbm.at[pl.ds(start + q * CHUNK,
                                                     CHUNK)])


@functools.partial(
    pl.kernel,
    out_type=jax.ShapeDtypeStruct((NC, NPAD, H), _f32),
    mesh=_mesh,
    scratch_types=[
        pltpu.VMEM((ECH_T, EC), jnp.int32),
        pltpu.VMEM((EC,), jnp.int32),
        pltpu.VMEM((EC,), jnp.int32),
        pltpu.VMEM((EC,), jnp.int32),
        pltpu.VMEM((EC,), jnp.int32),
        pltpu.VMEM((EC, H), _f32),
        pltpu.VMEM((EC, H), _f32),
        pltpu.VMEM_SHARED((NPAD, H), _f32),
        pltpu.SemaphoreType.DMA,
        pltpu.SemaphoreType.DMA,
    ],
)
def _sc_prop(y_hbm, pidx_hbm, out_hbm, pidx, sia, sib, dia, dib, bufa, bufb,
             acc, sema, semb):
    c = lax.axis_index("c")
    s = lax.axis_index("s")
    start = s * ROWS_T
    wid = s * NC + c
    base = wid * ECH_T

    pltpu.sync_copy(pidx_hbm.at[pl.ds(base, ECH_T)], pidx)

    def unpack(k, sbuf, dbuf):
        for v in range(EC // 16 + 1):
            off = min(v * 16, EC - 16)
            p = pidx[k, pl.ds(off, 16)]
            dbuf[pl.ds(off, 16)] = lax.shift_right_logical(p, 16)
            sbuf[pl.ds(off, 16)] = lax.bitwise_and(p, 0xFFFF)

    def zrow(r, carry):
        for k in range(H // 16):
            bufa[r, pl.ds(k * 16, 16)] = jnp.zeros((16,), _f32)
        return carry

    lax.fori_loop(0, EC, zrow, 0)
    zds = [pltpu.async_copy(bufa, acc.at[pl.ds(start + q * EC, EC)],
                            sema) for q in range(ROWS_T // EC)]
    zds.append(pltpu.async_copy(
        bufa.at[pl.ds(0, ROWS_T - EC * (ROWS_T // EC))],
        acc.at[pl.ds(start + EC * (ROWS_T // EC),
                     ROWS_T - EC * (ROWS_T // EC))], sema))
    for zd in zds:
        zd.wait()
    plsc.subcore_barrier()

    unpack(0, sia, dia)
    pltpu.async_copy(y_hbm.at[sia], bufa, sema).wait()

    def pair(j, carry):
        c0 = 2 * j
        unpack(c0 + 1, sib, dib)
        db = pltpu.async_copy(y_hbm.at[sib], bufb, semb)
        pltpu.sync_copy(bufa, acc.at[dia], add=True)
        db.wait()

        @pl.when(j < ECH_T // 2 - 1)
        def _():
            unpack(c0 + 2, sia, dia)
            da = pltpu.async_copy(y_hbm.at[sia], bufa, sema)
            pltpu.sync_copy(bufb, acc.at[dib], add=True)
            da.wait()

        @pl.when(j == ECH_T // 2 - 1)
        def _():
            pltpu.sync_copy(bufb, acc.at[dib], add=True)

        return carry

    lax.fori_loop(0, ECH_T // 2, pair, 0)
    plsc.subcore_barrier()
    pltpu.sync_copy(acc.at[pl.ds(start, ROWS_T)],
                    out_hbm.at[c, pl.ds(start, ROWS_T)])


def _pre_body(d0_ref, d1_ref, z_ref, dv_ref, y0_ref):
    deg = d0_ref[...] + d1_ref[...] + 1.0
    dv = jnp.transpose(lax.rsqrt(jnp.maximum(deg, 1.0)))
    dv_ref[...] = dv
    y0_ref[...] = z_ref[...] * dv[:N]


def _tc_pre(deg0, deg1, pearl_z):
    return pl.pallas_call(
        _pre_body,
        out_shape=[jax.ShapeDtypeStruct((NPAD, 1), _f32),
                   jax.ShapeDtypeStruct((N, H), _f32)],
    )(deg0.reshape(1, NPAD), deg1.reshape(1, NPAD), pearl_z)


def _dot(a, b):
    return jnp.dot(a, b, preferred_element_type=_f32)


def _s1_body(p_ref, y_ref, dv_ref, w_ref, b_ref, o_ref):
    dv = dv_ref[...]
    p = p_ref[...]
    u = (p[0] + p[1] + y_ref[...]) * dv
    z = jnp.maximum(_dot(u, w_ref[...]) + b_ref[...], 0.0)
    o_ref[...] = z * dv


def _tc_stage1(p, y, dinv_col, w, b):
    return pl.pallas_call(
        _s1_body,
        grid=(G,),
        in_specs=[pl.BlockSpec((NC, R, H), lambda i: (0, i, 0)),
                  pl.BlockSpec((R, H), lambda i: (i, 0)),
                  pl.BlockSpec((R, 1), lambda i: (i, 0)),
                  pl.BlockSpec((H, H), lambda i: (0, 0)),
                  pl.BlockSpec((1, H), lambda i: (0, 0))],
        out_specs=pl.BlockSpec((R, H), lambda i: (i, 0)),
        out_shape=jax.ShapeDtypeStruct((N, H), _f32),
    )(p, y, dinv_col, w, b)


def _s2_body(p_ref, y_ref, dv_ref, x_ref, wp2_ref, bp2_ref, wpo_ref, bpo_ref,
             wina_ref, winb_ref, bin_ref, wc1_ref, o_ref):
    dv = dv_ref[...]
    p = p_ref[...]
    u = (p[0] + p[1] + y_ref[...]) * dv
    z2 = jnp.maximum(_dot(u, wp2_ref[...]) + bp2_ref[...], 0.0)
    pe = _dot(z2, wpo_ref[...]) + bpo_ref[...]
    h0 = jnp.maximum(_dot(x_ref[...], wina_ref[...]) + _dot(pe, winb_ref[...])
                     + bin_ref[...], 0.0)
    t1 = _dot(h0, wc1_ref[...])
    o_ref[...] = t1 * dv


def _tc_stage2(p, y, dinv_col, x, wp2, bp2, wpo, bpo, wina, winb, b_in, wc1):
    return pl.pallas_call(
        _s2_body,
        grid=(G,),
        in_specs=[pl.BlockSpec((NC, R, H), lambda i: (0, i, 0)),
                  pl.BlockSpec((R, H), lambda i: (i, 0)),
                  pl.BlockSpec((R, 1), lambda i: (i, 0)),
                  pl.BlockSpec((R, D), lambda i: (i, 0)),
                  pl.BlockSpec((H, H), lambda i: (0, 0)),
                  pl.BlockSpec((1, H), lambda i: (0, 0)),
                  pl.BlockSpec((H, PE), lambda i: (0, 0)),
                  pl.BlockSpec((1, PE), lambda i: (0, 0)),
                  pl.BlockSpec((D, H), lambda i: (0, 0)),
                  pl.BlockSpec((PE, H), lambda i: (0, 0)),
                  pl.BlockSpec((1, H), lambda i: (0, 0)),
                  pl.BlockSpec((H, H), lambda i: (0, 0))],
        out_specs=pl.BlockSpec((R, H), lambda i: (i, 0)),
        out_shape=jax.ShapeDtypeStruct((N, H), _f32),
    )(p, y, dinv_col, x, wp2, bp2, wpo, bpo, wina, winb, b_in, wc1)


def _mid_body(residual, p_ref, y_ref, dv_ref, bc_ref, g_ref, be_ref, w_ref,
              *rest):
    if residual:
        hprev_ref, oy_ref, oh_ref = rest
    else:
        oy_ref, oh_ref = rest
    dv = dv_ref[...]
    p = p_ref[...]
    u = (p[0] + p[1] + y_ref[...]) * dv
    h = jnp.maximum((u + bc_ref[...]) * (g_ref[...] * BN_SCALE) + be_ref[...],
                    0.0)
    if residual:
        h = h + hprev_ref[...]
    oh_ref[...] = h
    oy_ref[...] = _dot(h, w_ref[...]) * dv


def _tc_mid(p, y, dinv_col, bc, g, be, w, hprev=None):
    residual = hprev is not None
    specs = [pl.BlockSpec((NC, R, H), lambda i: (0, i, 0)),
             pl.BlockSpec((R, H), lambda i: (i, 0)),
             pl.BlockSpec((R, 1), lambda i: (i, 0)),
             pl.BlockSpec((1, H), lambda i: (0, 0)),
             pl.BlockSpec((1, H), lambda i: (0, 0)),
             pl.BlockSpec((1, H), lambda i: (0, 0)),
             pl.BlockSpec((H, H), lambda i: (0, 0))]
    args = [p, y, dinv_col, bc, g, be, w]
    if residual:
        specs.append(pl.BlockSpec((R, H), lambda i: (i, 0)))
        args.append(hprev)
    return pl.pallas_call(
        functools.partial(_mid_body, residual),
        grid=(G,),
        in_specs=specs,
        out_specs=[pl.BlockSpec((R, H), lambda i: (i, 0)),
                   pl.BlockSpec((R, H), lambda i: (i, 0))],
        out_shape=[jax.ShapeDtypeStruct((N, H), _f32),
                   jax.ShapeDtypeStruct((N, H), _f32)],
    )(*args)


def _s5_body(p_ref, y_ref, dv_ref, bc_ref, g_ref, be_ref, hprev_ref,
             wout_ref, bout_ref, o_ref, acc_ref):
    i = pl.program_id(0)
    dv = dv_ref[...]
    p = p_ref[...]
    u = (p[0] + p[1] + y_ref[...]) * dv
    h = jnp.maximum((u + bc_ref[...]) * (g_ref[...] * BN_SCALE) + be_ref[...],
                    0.0) + hprev_ref[...]
    part = jnp.sum(h, axis=0, keepdims=True)

    @pl.when(i == 0)
    def _():
        acc_ref[...] = part

    @pl.when(i > 0)
    def _():
        acc_ref[...] = acc_ref[...] + part

    @pl.when(i == G - 1)
    def _():
        o_ref[...] = _dot(acc_ref[...] * (1.0 / N), wout_ref[...]) \
            + bout_ref[...]


def _tc_stage5(p, y, dinv_col, bc, g, be, hprev, wout, bout):
    return pl.pallas_call(
        _s5_body,
        grid=(G,),
        in_specs=[pl.BlockSpec((NC, R, H), lambda i: (0, i, 0)),
                  pl.BlockSpec((R, H), lambda i: (i, 0)),
                  pl.BlockSpec((R, 1), lambda i: (i, 0)),
                  pl.BlockSpec((1, H), lambda i: (0, 0)),
                  pl.BlockSpec((1, H), lambda i: (0, 0)),
                  pl.BlockSpec((1, H), lambda i: (0, 0)),
                  pl.BlockSpec((R, H), lambda i: (i, 0)),
                  pl.BlockSpec((H, OUT), lambda i: (0, 0)),
                  pl.BlockSpec((1, OUT), lambda i: (0, 0))],
        out_specs=pl.BlockSpec((1, OUT), lambda i: (0, 0)),
        out_shape=jax.ShapeDtypeStruct((1, OUT), _f32),
        scratch_shapes=[pltpu.VMEM((1, H), _f32)],
    )(p, y, dinv_col, bc, g, be, hprev, wout, bout)


def kernel(x, edge_index, pearl_z, Wp1, bp1, Wp2, bp2, Wpo, bpo, Win, b_in,
           Wc1, bc1, Wc2, bc2, Wc3, bc3, g1, be1, g2, be2, g3, be3,
           Wout, bout):
    src = edge_index[0]
    dst = edge_index[1]
    dst2d = dst.reshape(E // EC, EC)
    packed2d = ((dst << 16) | src).reshape(E // EC, EC)
    row = lambda v: v.reshape(1, -1)

    deg0, deg1 = _sc_degree(dst2d)
    dinv_col, y0 = _tc_pre(deg0, deg1, pearl_z)
    p = _sc_prop(y0, packed2d)
    y1 = _tc_stage1(p, y0, dinv_col, Wp1, row(bp1))
    p = _sc_prop(y1, packed2d)
    y2 = _tc_stage2(p, y1, dinv_col, x, Wp2, row(bp2), Wpo, row(bpo),
                    Win[:D], Win[D:], row(b_in), Wc1)
    p = _sc_prop(y2, packed2d)
    y3, h1 = _tc_mid(p, y2, dinv_col, row(bc1), row(g1), row(be1), Wc2)
    p = _sc_prop(y3, packed2d)
    y4, h2 = _tc_mid(p, y3, dinv_col, row(bc2), row(g2), row(be2), Wc3,
                     hprev=h1)
    p = _sc_prop(y4, packed2d)
    return _tc_stage5(p, y4, dinv_col, row(bc3), row(g3), row(be3), h2,
                      Wout, row(bout))


# speedup vs baseline: 1.0214x; 1.0214x over previous
import functools
import math

import jax
import jax.numpy as jnp
from jax import lax
from jax.experimental import pallas as pl
from jax.experimental.pallas import tpu as pltpu
from jax.experimental.pallas import tpu_sc as plsc

N = 10000
E = 320000
D = 128
H = 128
PE = 32
OUT = 10

NC = 2
NS = 16
NW = NC * NS
CHUNK = 128
ROWS_T = 640
NPAD = NS * ROWS_T
EC = 125
EC8 = 128
ECH_T = (E // NW) // EC

R = 2000
G = N // R
EPS = 1e-5
BN_SCALE = 1.0 / math.sqrt(1.0 + EPS)

_f32 = jnp.float32
_mesh = plsc.VectorSubcoreMesh(core_axis_name="c", subcore_axis_name="s",
                               num_cores=NC, num_subcores=NS)


@functools.partial(
    pl.kernel,
    out_type=[jax.ShapeDtypeStruct((NPAD,), _f32),
              jax.ShapeDtypeStruct((NPAD,), _f32)],
    mesh=_mesh,
    scratch_types=[
        pltpu.VMEM((EC,), jnp.int32),
        pltpu.VMEM((EC,), jnp.int32),
        pltpu.VMEM((EC8,), _f32),
        pltpu.VMEM((CHUNK,), _f32),
        pltpu.VMEM_SHARED((NPAD,), _f32),
        pltpu.SemaphoreType.DMA,
        pltpu.SemaphoreType.DMA,
    ],
)
def _sc_degree(dst_hbm, out0_hbm, out1_hbm, dia, dib, ones, zf, acc,
               dsema, dsemb):
    c = lax.axis_index("c")
    s = lax.axis_index("s")
    start = s * ROWS_T
    wid = s * NC + c
    base = wid * ECH_T
    for k in range(EC8 // 16):
        ones[pl.ds(k * 16, 16)] = jnp.ones((16,), _f32)
    for k in range(CHUNK // 16):
        zf[pl.ds(k * 16, 16)] = jnp.zeros((16,), _f32)
    zds = [pltpu.async_copy(zf, acc.at[pl.ds(start + q * CHUNK, CHUNK)],
                            dsema) for q in range(ROWS_T // CHUNK)]
    for zd in zds:
        zd.wait()
    plsc.subcore_barrier()

    pltpu.sync_copy(dst_hbm.at[base], dia)

    def pair(j, carry):
        c0 = 2 * j
        ddb = pltpu.async_copy(dst_hbm.at[base + c0 + 1], dib, dsemb)
        pltpu.sync_copy(ones.at[pl.ds(0, EC)], acc.at[dia], add=True)
        ddb.wait()

        @pl.when(j < ECH_T // 2 - 1)
        def _():
            dda = pltpu.async_copy(dst_hbm.at[base + c0 + 2], dia, dsema)
            pltpu.sync_copy(ones.at[pl.ds(0, EC)], acc.at[dib], add=True)
            dda.wait()

        @pl.when(j == ECH_T // 2 - 1)
        def _():
            pltpu.sync_copy(ones.at[pl.ds(0, EC)], acc.at[dib], add=True)

        return carry

    lax.fori_loop(0, ECH_T // 2, pair, 0)
    plsc.subcore_barrier()
    for cc, out_hbm in ((0, out0_hbm), (1, out1_hbm)):
        @pl.when(c == cc)
        def _():
            for q in range(ROWS_T // CHUNK):
                pltpu.sync_copy(acc.at[pl.ds(start + q * CHUNK, CHUNK)], zf)
                pltpu.sync_copy(zf, out_hbm.at[pl.ds(start + q * CHUNK,
                                                     CHUNK)])


@functools.partial(
    pl.kernel,
    out_type=jax.ShapeDtypeStruct((NC, NPAD, H), _f32),
    mesh=_mesh,
    scratch_types=[
        pltpu.VMEM((ECH_T, EC), jnp.int32),
        pltpu.VMEM((EC,), jnp.int32),
        pltpu.VMEM((EC,), jnp.int32),
        pltpu.VMEM((CHUNK, H), _f32),
        pltpu.VMEM((CHUNK, H), _f32),
        pltpu.VMEM_SHARED((NPAD, H), _f32),
        pltpu.SemaphoreType.DMA,
        pltpu.SemaphoreType.DMA,
        pltpu.SemaphoreType.DMA,
        pltpu.SemaphoreType.DMA,
    ],
)
def _sc_prop(y_hbm, src_hbm, dst_hbm, out_hbm, sidx, dia, dib, bufa, bufb,
             acc, sema, semb, dsema, dsemb):
    c = lax.axis_index("c")
    s = lax.axis_index("s")
    start = s * ROWS_T
    wid = s * NC + c
    base = wid * ECH_T

    pltpu.sync_copy(src_hbm.at[pl.ds(base, ECH_T)], sidx)

    def zrow(r, carry):
        for k in range(H // 16):
            bufa[r, pl.ds(k * 16, 16)] = jnp.zeros((16,), _f32)
        return carry

    lax.fori_loop(0, CHUNK, zrow, 0)
    zds = [pltpu.async_copy(bufa, acc.at[pl.ds(start + q * CHUNK, CHUNK)],
                            sema) for q in range(ROWS_T // CHUNK)]
    for zd in zds:
        zd.wait()
    plsc.subcore_barrier()

    pltpu.sync_copy(dst_hbm.at[base], dia)
    pltpu.async_copy(y_hbm.at[sidx.at[0]], bufa.at[pl.ds(0, EC)], sema).wait()

    def pair(j, carry):
        c0 = 2 * j
        ddb = pltpu.async_copy(dst_hbm.at[base + c0 + 1], dib, dsemb)
        db = pltpu.async_copy(y_hbm.at[sidx.at[c0 + 1]],
                              bufb.at[pl.ds(0, EC)], semb)
        pltpu.sync_copy(bufa.at[pl.ds(0, EC)], acc.at[dia], add=True)
        ddb.wait()
        db.wait()

        @pl.when(j < ECH_T // 2 - 1)
        def _():
            dda = pltpu.async_copy(dst_hbm.at[base + c0 + 2], dia, dsema)
            da = pltpu.async_copy(y_hbm.at[sidx.at[c0 + 2]],
                                  bufa.at[pl.ds(0, EC)], sema)
            pltpu.sync_copy(bufb.at[pl.ds(0, EC)], acc.at[dib], add=True)
            dda.wait()
            da.wait()

        @pl.when(j == ECH_T // 2 - 1)
        def _():
            pltpu.sync_copy(bufb.at[pl.ds(0, EC)], acc.at[dib], add=True)

        return carry

    lax.fori_loop(0, ECH_T // 2, pair, 0)
    plsc.subcore_barrier()
    pltpu.sync_copy(acc.at[pl.ds(start, ROWS_T)],
                    out_hbm.at[c, pl.ds(start, ROWS_T)])


def _pre_body(d0_ref, d1_ref, z_ref, dv_ref, y0_ref):
    deg = d0_ref[...] + d1_ref[...] + 1.0
    dv = jnp.transpose(lax.rsqrt(jnp.maximum(deg, 1.0)))
    dv_ref[...] = dv
    y0_ref[...] = z_ref[...] * dv[:N]


def _tc_pre(deg0, deg1, pearl_z):
    return pl.pallas_call(
        _pre_body,
        out_shape=[jax.ShapeDtypeStruct((NPAD, 1), _f32),
                   jax.ShapeDtypeStruct((N, H), _f32)],
    )(deg0.reshape(1, NPAD), deg1.reshape(1, NPAD), pearl_z)


def _dot(a, b):
    return jnp.dot(a, b, preferred_element_type=_f32)


def _s1_body(p_ref, y_ref, dv_ref, w_ref, b_ref, o_ref):
    dv = dv_ref[...]
    p = p_ref[...]
    u = (p[0] + p[1] + y_ref[...]) * dv
    z = jnp.maximum(_dot(u, w_ref[...]) + b_ref[...], 0.0)
    o_ref[...] = z * dv


def _tc_stage1(p, y, dinv_col, w, b):
    return pl.pallas_call(
        _s1_body,
        grid=(G,),
        in_specs=[pl.BlockSpec((NC, R, H), lambda i: (0, i, 0)),
                  pl.BlockSpec((R, H), lambda i: (i, 0)),
                  pl.BlockSpec((R, 1), lambda i: (i, 0)),
                  pl.BlockSpec((H, H), lambda i: (0, 0)),
                  pl.BlockSpec((1, H), lambda i: (0, 0))],
        out_specs=pl.BlockSpec((R, H), lambda i: (i, 0)),
        out_shape=jax.ShapeDtypeStruct((N, H), _f32),
    )(p, y, dinv_col, w, b)


def _s2_body(p_ref, y_ref, dv_ref, x_ref, wp2_ref, bp2_ref, wpo_ref, bpo_ref,
             wina_ref, winb_ref, bin_ref, wc1_ref, o_ref):
    dv = dv_ref[...]
    p = p_ref[...]
    u = (p[0] + p[1] + y_ref[...]) * dv
    z2 = jnp.maximum(_dot(u, wp2_ref[...]) + bp2_ref[...], 0.0)
    pe = _dot(z2, wpo_ref[...]) + bpo_ref[...]
    h0 = jnp.maximum(_dot(x_ref[...], wina_ref[...]) + _dot(pe, winb_ref[...])
                     + bin_ref[...], 0.0)
    t1 = _dot(h0, wc1_ref[...])
    o_ref[...] = t1 * dv


def _tc_stage2(p, y, dinv_col, x, wp2, bp2, wpo, bpo, wina, winb, b_in, wc1):
    return pl.pallas_call(
        _s2_body,
        grid=(G,),
        in_specs=[pl.BlockSpec((NC, R, H), lambda i: (0, i, 0)),
                  pl.BlockSpec((R, H), lambda i: (i, 0)),
                  pl.BlockSpec((R, 1), lambda i: (i, 0)),
                  pl.BlockSpec((R, D), lambda i: (i, 0)),
                  pl.BlockSpec((H, H), lambda i: (0, 0)),
                  pl.BlockSpec((1, H), lambda i: (0, 0)),
                  pl.BlockSpec((H, PE), lambda i: (0, 0)),
                  pl.BlockSpec((1, PE), lambda i: (0, 0)),
                  pl.BlockSpec((D, H), lambda i: (0, 0)),
                  pl.BlockSpec((PE, H), lambda i: (0, 0)),
                  pl.BlockSpec((1, H), lambda i: (0, 0)),
                  pl.BlockSpec((H, H), lambda i: (0, 0))],
        out_specs=pl.BlockSpec((R, H), lambda i: (i, 0)),
        out_shape=jax.ShapeDtypeStruct((N, H), _f32),
    )(p, y, dinv_col, x, wp2, bp2, wpo, bpo, wina, winb, b_in, wc1)


def _mid_body(residual, p_ref, y_ref, dv_ref, bc_ref, g_ref, be_ref, w_ref,
              *rest):
    if residual:
        hprev_ref, oy_ref, oh_ref = rest
    else:
        oy_ref, oh_ref = rest
    dv = dv_ref[...]
    p = p_ref[...]
    u = (p[0] + p[1] + y_ref[...]) * dv
    h = jnp.maximum((u + bc_ref[...]) * (g_ref[...] * BN_SCALE) + be_ref[...],
                    0.0)
    if residual:
        h = h + hprev_ref[...]
    oh_ref[...] = h
    oy_ref[...] = _dot(h, w_ref[...]) * dv


def _tc_mid(p, y, dinv_col, bc, g, be, w, hprev=None):
    residual = hprev is not None
    specs = [pl.BlockSpec((NC, R, H), lambda i: (0, i, 0)),
             pl.BlockSpec((R, H), lambda i: (i, 0)),
             pl.BlockSpec((R, 1), lambda i: (i, 0)),
             pl.BlockSpec((1, H), lambda i: (0, 0)),
             pl.BlockSpec((1, H), lambda i: (0, 0)),
             pl.BlockSpec((1, H), lambda i: (0, 0)),
             pl.BlockSpec((H, H), lambda i: (0, 0))]
    args = [p, y, dinv_col, bc, g, be, w]
    if residual:
        specs.append(pl.BlockSpec((R, H), lambda i: (i, 0)))
        args.append(hprev)
    return pl.pallas_call(
        functools.partial(_mid_body, residual),
        grid=(G,),
        in_specs=specs,
        out_specs=[pl.BlockSpec((R, H), lambda i: (i, 0)),
                   pl.BlockSpec((R, H), lambda i: (i, 0))],
        out_shape=[jax.ShapeDtypeStruct((N, H), _f32),
                   jax.ShapeDtypeStruct((N, H), _f32)],
    )(*args)


def _s5_body(p_ref, y_ref, dv_ref, bc_ref, g_ref, be_ref, hprev_ref,
             wout_ref, bout_ref, o_ref, acc_ref):
    i = pl.program_id(0)
    dv = dv_ref[...]
    p = p_ref[...]
    u = (p[0] + p[1] + y_ref[...]) * dv
    h = jnp.maximum((u + bc_ref[...]) * (g_ref[...] * BN_SCALE) + be_ref[...],
                    0.0) + hprev_ref[...]
    part = jnp.sum(h, axis=0, keepdims=True)

    @pl.when(i == 0)
    def _():
        acc_ref[...] = part

    @pl.when(i > 0)
    def _():
        acc_ref[...] = acc_ref[...] + part

    @pl.when(i == G - 1)
    def _():
        o_ref[...] = _dot(acc_ref[...] * (1.0 / N), wout_ref[...]) \
            + bout_ref[...]


def _tc_stage5(p, y, dinv_col, bc, g, be, hprev, wout, bout):
    return pl.pallas_call(
        _s5_body,
        grid=(G,),
        in_specs=[pl.BlockSpec((NC, R, H), lambda i: (0, i, 0)),
                  pl.BlockSpec((R, H), lambda i: (i, 0)),
                  pl.BlockSpec((R, 1), lambda i: (i, 0)),
                  pl.BlockSpec((1, H), lambda i: (0, 0)),
                  pl.BlockSpec((1, H), lambda i: (0, 0)),
                  pl.BlockSpec((1, H), lambda i: (0, 0)),
                  pl.BlockSpec((R, H), lambda i: (i, 0)),
                  pl.BlockSpec((H, OUT), lambda i: (0, 0)),
                  pl.BlockSpec((1, OUT), lambda i: (0, 0))],
        out_specs=pl.BlockSpec((1, OUT), lambda i: (0, 0)),
        out_shape=jax.ShapeDtypeStruct((1, OUT), _f32),
        scratch_shapes=[pltpu.VMEM((1, H), _f32)],
    )(p, y, dinv_col, bc, g, be, hprev, wout, bout)


def kernel(x, edge_index, pearl_z, Wp1, bp1, Wp2, bp2, Wpo, bpo, Win, b_in,
           Wc1, bc1, Wc2, bc2, Wc3, bc3, g1, be1, g2, be2, g3, be3,
           Wout, bout):
    src = edge_index[0]
    dst = edge_index[1]
    src2d = src.reshape(E // EC, EC)
    dst2d = dst.reshape(E // EC, EC)
    row = lambda v: v.reshape(1, -1)

    deg0, deg1 = _sc_degree(dst2d)
    dinv_col, y0 = _tc_pre(deg0, deg1, pearl_z)
    p = _sc_prop(y0, src2d, dst2d)
    y1 = _tc_stage1(p, y0, dinv_col, Wp1, row(bp1))
    p = _sc_prop(y1, src2d, dst2d)
    y2 = _tc_stage2(p, y1, dinv_col, x, Wp2, row(bp2), Wpo, row(bpo),
                    Win[:D], Win[D:], row(b_in), Wc1)
    p = _sc_prop(y2, src2d, dst2d)
    y3, h1 = _tc_mid(p, y2, dinv_col, row(bc1), row(g1), row(be1), Wc2)
    p = _sc_prop(y3, src2d, dst2d)
    y4, h2 = _tc_mid(p, y3, dinv_col, row(bc2), row(g2), row(be2), Wc3,
                     hprev=h1)
    p = _sc_prop(y4, src2d, dst2d)
    return _tc_stage5(p, y4, dinv_col, row(bc3), row(g3), row(be3), h2,
                      Wout, row(bout))


# speedup vs baseline: 1.0232x; 1.0018x over previous
import functools
import math

import jax
import jax.numpy as jnp
from jax import lax
from jax.experimental import pallas as pl
from jax.experimental.pallas import tpu as pltpu
from jax.experimental.pallas import tpu_sc as plsc

N = 10000
E = 320000
D = 128
H = 128
PE = 32
OUT = 10

NC = 2
NS = 16
NW = NC * NS
CHUNK = 128
ROWS_T = 640
NPAD = NS * ROWS_T
EC = 125
EC8 = 128
ECH_T = (E // NW) // EC

R = 2000
G = N // R
EPS = 1e-5
BN_SCALE = 1.0 / math.sqrt(1.0 + EPS)

_f32 = jnp.float32
_mesh = plsc.VectorSubcoreMesh(core_axis_name="c", subcore_axis_name="s",
                               num_cores=NC, num_subcores=NS)


@functools.partial(
    pl.kernel,
    out_type=[jax.ShapeDtypeStruct((NPAD,), _f32),
              jax.ShapeDtypeStruct((NPAD,), _f32)],
    mesh=_mesh,
    scratch_types=[
        pltpu.VMEM((4, EC), jnp.int32),
        pltpu.VMEM((EC8,), _f32),
        pltpu.VMEM((CHUNK,), _f32),
        pltpu.VMEM_SHARED((NPAD,), _f32),
        pltpu.SemaphoreType.DMA,
        pltpu.SemaphoreType.DMA,
    ],
)
def _sc_degree(dst_hbm, out0_hbm, out1_hbm, dix, ones, zf, acc,
               dsema, ssem):
    c = lax.axis_index("c")
    s = lax.axis_index("s")
    start = s * ROWS_T
    wid = s * NC + c
    base = wid * ECH_T
    for k in range(EC8 // 16):
        ones[pl.ds(k * 16, 16)] = jnp.ones((16,), _f32)
    for k in range(CHUNK // 16):
        zf[pl.ds(k * 16, 16)] = jnp.zeros((16,), _f32)
    zds = [pltpu.async_copy(zf, acc.at[pl.ds(start + q * CHUNK, CHUNK)],
                            dsema) for q in range(ROWS_T // CHUNK)]
    for zd in zds:
        zd.wait()
    plsc.subcore_barrier()

    def quad(q, carry):
        sds = []
        for k in range(4):
            pltpu.sync_copy(dst_hbm.at[base + 4 * q + k], dix.at[k])
            sds.append(pltpu.async_copy(ones.at[pl.ds(0, EC)],
                                        acc.at[dix.at[k]], ssem, add=True))
        for sd in sds:
            sd.wait()
        return carry

    lax.fori_loop(0, ECH_T // 4, quad, 0)
    plsc.subcore_barrier()
    for cc, out_hbm in ((0, out0_hbm), (1, out1_hbm)):
        @pl.when(c == cc)
        def _():
            for q in range(ROWS_T // CHUNK):
                pltpu.sync_copy(acc.at[pl.ds(start + q * CHUNK, CHUNK)], zf)
                pltpu.sync_copy(zf, out_hbm.at[pl.ds(start + q * CHUNK,
                                                     CHUNK)])


@functools.partial(
    pl.kernel,
    out_type=jax.ShapeDtypeStruct((NC, NPAD, H), _f32),
    mesh=_mesh,
    scratch_types=[
        pltpu.VMEM((ECH_T, EC), jnp.int32),
        pltpu.VMEM((EC,), jnp.int32),
        pltpu.VMEM((EC,), jnp.int32),
        pltpu.VMEM((CHUNK, H), _f32),
        pltpu.VMEM((CHUNK, H), _f32),
        pltpu.VMEM_SHARED((NPAD, H), _f32),
        pltpu.SemaphoreType.DMA,
        pltpu.SemaphoreType.DMA,
        pltpu.SemaphoreType.DMA,
        pltpu.SemaphoreType.DMA,
    ],
)
def _sc_prop(y_hbm, src_hbm, dst_hbm, out_hbm, sidx, dia, dib, bufa, bufb,
             acc, sema, semb, dsema, dsemb):
    c = lax.axis_index("c")
    s = lax.axis_index("s")
    start = s * ROWS_T
    wid = s * NC + c
    base = wid * ECH_T

    pltpu.sync_copy(src_hbm.at[pl.ds(base, ECH_T)], sidx)

    def zrow(r, carry):
        for k in range(H // 16):
            bufa[r, pl.ds(k * 16, 16)] = jnp.zeros((16,), _f32)
        return carry

    lax.fori_loop(0, CHUNK, zrow, 0)
    zds = [pltpu.async_copy(bufa, acc.at[pl.ds(start + q * CHUNK, CHUNK)],
                            dsema) for q in range(ROWS_T // CHUNK)]
    pltpu.sync_copy(dst_hbm.at[base], dia)
    prime = pltpu.async_copy(y_hbm.at[sidx.at[0]], bufb.at[pl.ds(0, EC)],
                             semb)
    for zd in zds:
        zd.wait()
    plsc.subcore_barrier()
    prime.wait()

    def pair(j, carry):
        c0 = 2 * j
        ddb = pltpu.async_copy(dst_hbm.at[base + c0 + 1], dib, dsemb)
        da = pltpu.async_copy(y_hbm.at[sidx.at[c0 + 1]],
                              bufa.at[pl.ds(0, EC)], sema)
        pltpu.sync_copy(bufb.at[pl.ds(0, EC)], acc.at[dia], add=True)
        ddb.wait()
        da.wait()

        @pl.when(j < ECH_T // 2 - 1)
        def _():
            dda = pltpu.async_copy(dst_hbm.at[base + c0 + 2], dia, dsema)
            db = pltpu.async_copy(y_hbm.at[sidx.at[c0 + 2]],
                                  bufb.at[pl.ds(0, EC)], semb)
            pltpu.sync_copy(bufa.at[pl.ds(0, EC)], acc.at[dib], add=True)
            dda.wait()
            db.wait()

        @pl.when(j == ECH_T // 2 - 1)
        def _():
            pltpu.sync_copy(bufa.at[pl.ds(0, EC)], acc.at[dib], add=True)

        return carry

    lax.fori_loop(0, ECH_T // 2, pair, 0)
    plsc.subcore_barrier()
    pltpu.sync_copy(acc.at[pl.ds(start, ROWS_T)],
                    out_hbm.at[c, pl.ds(start, ROWS_T)])


def _pre_body(d0_ref, d1_ref, z_ref, dv_ref, y0_ref):
    deg = d0_ref[...] + d1_ref[...] + 1.0
    dv = jnp.transpose(lax.rsqrt(jnp.maximum(deg, 1.0)))
    dv_ref[...] = dv
    y0_ref[...] = z_ref[...] * dv[:N]


def _tc_pre(deg0, deg1, pearl_z):
    return pl.pallas_call(
        _pre_body,
        out_shape=[jax.ShapeDtypeStruct((NPAD, 1), _f32),
                   jax.ShapeDtypeStruct((N, H), _f32)],
    )(deg0.reshape(1, NPAD), deg1.reshape(1, NPAD), pearl_z)


def _dot(a, b):
    return jnp.dot(a, b, preferred_element_type=_f32)


def _s1_body(p_ref, y_ref, dv_ref, w_ref, b_ref, o_ref):
    dv = dv_ref[...]
    p = p_ref[...]
    u = (p[0] + p[1] + y_ref[...]) * dv
    z = jnp.maximum(_dot(u, w_ref[...]) + b_ref[...], 0.0)
    o_ref[...] = z * dv


def _tc_stage1(p, y, dinv_col, w, b):
    return pl.pallas_call(
        _s1_body,
        grid=(G,),
        in_specs=[pl.BlockSpec((NC, R, H), lambda i: (0, i, 0)),
                  pl.BlockSpec((R, H), lambda i: (i, 0)),
                  pl.BlockSpec((R, 1), lambda i: (i, 0)),
                  pl.BlockSpec((H, H), lambda i: (0, 0)),
                  pl.BlockSpec((1, H), lambda i: (0, 0))],
        out_specs=pl.BlockSpec((R, H), lambda i: (i, 0)),
        out_shape=jax.ShapeDtypeStruct((N, H), _f32),
    )(p, y, dinv_col, w, b)


def _s2_body(p_ref, y_ref, dv_ref, x_ref, wp2_ref, bp2_ref, wpo_ref, bpo_ref,
             wina_ref, winb_ref, bin_ref, wc1_ref, o_ref):
    dv = dv_ref[...]
    p = p_ref[...]
    u = (p[0] + p[1] + y_ref[...]) * dv
    z2 = jnp.maximum(_dot(u, wp2_ref[...]) + bp2_ref[...], 0.0)
    pe = _dot(z2, wpo_ref[...]) + bpo_ref[...]
    h0 = jnp.maximum(_dot(x_ref[...], wina_ref[...]) + _dot(pe, winb_ref[...])
                     + bin_ref[...], 0.0)
    t1 = _dot(h0, wc1_ref[...])
    o_ref[...] = t1 * dv


def _tc_stage2(p, y, dinv_col, x, wp2, bp2, wpo, bpo, wina, winb, b_in, wc1):
    return pl.pallas_call(
        _s2_body,
        grid=(G,),
        in_specs=[pl.BlockSpec((NC, R, H), lambda i: (0, i, 0)),
                  pl.BlockSpec((R, H), lambda i: (i, 0)),
                  pl.BlockSpec((R, 1), lambda i: (i, 0)),
                  pl.BlockSpec((R, D), lambda i: (i, 0)),
                  pl.BlockSpec((H, H), lambda i: (0, 0)),
                  pl.BlockSpec((1, H), lambda i: (0, 0)),
                  pl.BlockSpec((H, PE), lambda i: (0, 0)),
                  pl.BlockSpec((1, PE), lambda i: (0, 0)),
                  pl.BlockSpec((D, H), lambda i: (0, 0)),
                  pl.BlockSpec((PE, H), lambda i: (0, 0)),
                  pl.BlockSpec((1, H), lambda i: (0, 0)),
                  pl.BlockSpec((H, H), lambda i: (0, 0))],
        out_specs=pl.BlockSpec((R, H), lambda i: (i, 0)),
        out_shape=jax.ShapeDtypeStruct((N, H), _f32),
    )(p, y, dinv_col, x, wp2, bp2, wpo, bpo, wina, winb, b_in, wc1)


def _mid_body(residual, p_ref, y_ref, dv_ref, bc_ref, g_ref, be_ref, w_ref,
              *rest):
    if residual:
        hprev_ref, oy_ref, oh_ref = rest
    else:
        oy_ref, oh_ref = rest
    dv = dv_ref[...]
    p = p_ref[...]
    u = (p[0] + p[1] + y_ref[...]) * dv
    h = jnp.maximum((u + bc_ref[...]) * (g_ref[...] * BN_SCALE) + be_ref[...],
                    0.0)
    if residual:
        h = h + hprev_ref[...]
    oh_ref[...] = h
    oy_ref[...] = _dot(h, w_ref[...]) * dv


def _tc_mid(p, y, dinv_col, bc, g, be, w, hprev=None):
    residual = hprev is not None
    specs = [pl.BlockSpec((NC, R, H), lambda i: (0, i, 0)),
             pl.BlockSpec((R, H), lambda i: (i, 0)),
             pl.BlockSpec((R, 1), lambda i: (i, 0)),
             pl.BlockSpec((1, H), lambda i: (0, 0)),
             pl.BlockSpec((1, H), lambda i: (0, 0)),
             pl.BlockSpec((1, H), lambda i: (0, 0)),
             pl.BlockSpec((H, H), lambda i: (0, 0))]
    args = [p, y, dinv_col, bc, g, be, w]
    if residual:
        specs.append(pl.BlockSpec((R, H), lambda i: (i, 0)))
        args.append(hprev)
    return pl.pallas_call(
        functools.partial(_mid_body, residual),
        grid=(G,),
        in_specs=specs,
        out_specs=[pl.BlockSpec((R, H), lambda i: (i, 0)),
                   pl.BlockSpec((R, H), lambda i: (i, 0))],
        out_shape=[jax.ShapeDtypeStruct((N, H), _f32),
                   jax.ShapeDtypeStruct((N, H), _f32)],
    )(*args)


def _s5_body(p_ref, y_ref, dv_ref, bc_ref, g_ref, be_ref, hprev_ref,
             wout_ref, bout_ref, o_ref, acc_ref):
    i = pl.program_id(0)
    dv = dv_ref[...]
    p = p_ref[...]
    u = (p[0] + p[1] + y_ref[...]) * dv
    h = jnp.maximum((u + bc_ref[...]) * (g_ref[...] * BN_SCALE) + be_ref[...],
                    0.0) + hprev_ref[...]
    part = jnp.sum(h, axis=0, keepdims=True)

    @pl.when(i == 0)
    def _():
        acc_ref[...] = part

    @pl.when(i > 0)
    def _():
        acc_ref[...] = acc_ref[...] + part

    @pl.when(i == G - 1)
    def _():
        o_ref[...] = _dot(acc_ref[...] * (1.0 / N), wout_ref[...]) \
            + bout_ref[...]


def _tc_stage5(p, y, dinv_col, bc, g, be, hprev, wout, bout):
    return pl.pallas_call(
        _s5_body,
        grid=(G,),
        in_specs=[pl.BlockSpec((NC, R, H), lambda i: (0, i, 0)),
                  pl.BlockSpec((R, H), lambda i: (i, 0)),
                  pl.BlockSpec((R, 1), lambda i: (i, 0)),
                  pl.BlockSpec((1, H), lambda i: (0, 0)),
                  pl.BlockSpec((1, H), lambda i: (0, 0)),
                  pl.BlockSpec((1, H), lambda i: (0, 0)),
                  pl.BlockSpec((R, H), lambda i: (i, 0)),
                  pl.BlockSpec((H, OUT), lambda i: (0, 0)),
                  pl.BlockSpec((1, OUT), lambda i: (0, 0))],
        out_specs=pl.BlockSpec((1, OUT), lambda i: (0, 0)),
        out_shape=jax.ShapeDtypeStruct((1, OUT), _f32),
        scratch_shapes=[pltpu.VMEM((1, H), _f32)],
    )(p, y, dinv_col, bc, g, be, hprev, wout, bout)


def kernel(x, edge_index, pearl_z, Wp1, bp1, Wp2, bp2, Wpo, bpo, Win, b_in,
           Wc1, bc1, Wc2, bc2, Wc3, bc3, g1, be1, g2, be2, g3, be3,
           Wout, bout):
    src = edge_index[0]
    dst = edge_index[1]
    src2d = src.reshape(E // EC, EC)
    dst2d = dst.reshape(E // EC, EC)
    row = lambda v: v.reshape(1, -1)

    deg0, deg1 = _sc_degree(dst2d)
    dinv_col, y0 = _tc_pre(deg0, deg1, pearl_z)
    p = _sc_prop(y0, src2d, dst2d)
    y1 = _tc_stage1(p, y0, dinv_col, Wp1, row(bp1))
    p = _sc_prop(y1, src2d, dst2d)
    y2 = _tc_stage2(p, y1, dinv_col, x, Wp2, row(bp2), Wpo, row(bpo),
                    Win[:D], Win[D:], row(b_in), Wc1)
    p = _sc_prop(y2, src2d, dst2d)
    y3, h1 = _tc_mid(p, y2, dinv_col, row(bc1), row(g1), row(be1), Wc2)
    p = _sc_prop(y3, src2d, dst2d)
    y4, h2 = _tc_mid(p, y3, dinv_col, row(bc2), row(g2), row(be2), Wc3,
                     hprev=h1)
    p = _sc_prop(y4, src2d, dst2d)
    return _tc_stage5(p, y4, dinv_col, row(bc3), row(g3), row(be3), h2,
                      Wout, row(bout))


# speedup vs baseline: 1.0282x; 1.0049x over previous
import functools
import math

import jax
import jax.numpy as jnp
from jax import lax
from jax.experimental import pallas as pl
from jax.experimental.pallas import tpu as pltpu
from jax.experimental.pallas import tpu_sc as plsc

N = 10000
E = 320000
D = 128
H = 128
PE = 32
OUT = 10

NC = 2
NS = 16
NW = NC * NS
CHUNK = 128
ROWS_T = 640
NPAD = NS * ROWS_T
EC = 125
EC8 = 128
ECH_T = (E // NW) // EC

R = 2000
G = N // R
EPS = 1e-5
BN_SCALE = 1.0 / math.sqrt(1.0 + EPS)

_f32 = jnp.float32
_mesh = plsc.VectorSubcoreMesh(core_axis_name="c", subcore_axis_name="s",
                               num_cores=NC, num_subcores=NS)


@functools.partial(
    pl.kernel,
    out_type=[jax.ShapeDtypeStruct((NPAD,), _f32),
              jax.ShapeDtypeStruct((NPAD,), _f32)],
    mesh=_mesh,
    scratch_types=[
        pltpu.VMEM((4, EC), jnp.int32),
        pltpu.VMEM((EC8,), _f32),
        pltpu.VMEM((CHUNK,), _f32),
        pltpu.VMEM_SHARED((NPAD,), _f32),
        pltpu.SemaphoreType.DMA,
        pltpu.SemaphoreType.DMA,
    ],
)
def _sc_degree(dst_hbm, out0_hbm, out1_hbm, dix, ones, zf, acc,
               dsema, ssem):
    c = lax.axis_index("c")
    s = lax.axis_index("s")
    start = s * ROWS_T
    wid = s * NC + c
    base = wid * ECH_T
    for k in range(EC8 // 16):
        ones[pl.ds(k * 16, 16)] = jnp.ones((16,), _f32)
    for k in range(CHUNK // 16):
        zf[pl.ds(k * 16, 16)] = jnp.zeros((16,), _f32)
    zds = [pltpu.async_copy(zf, acc.at[pl.ds(start + q * CHUNK, CHUNK)],
                            dsema) for q in range(ROWS_T // CHUNK)]
    for zd in zds:
        zd.wait()
    plsc.subcore_barrier()

    def quad(q, carry):
        sds = []
        for k in range(4):
            pltpu.sync_copy(dst_hbm.at[base + 4 * q + k], dix.at[k])
            sds.append(pltpu.async_copy(ones.at[pl.ds(0, EC)],
                                        acc.at[dix.at[k]], ssem, add=True))
        for sd in sds:
            sd.wait()
        return carry

    lax.fori_loop(0, ECH_T // 4, quad, 0)
    plsc.subcore_barrier()
    for cc, out_hbm in ((0, out0_hbm), (1, out1_hbm)):
        @pl.when(c == cc)
        def _():
            for q in range(ROWS_T // CHUNK):
                pltpu.sync_copy(acc.at[pl.ds(start + q * CHUNK, CHUNK)], zf)
                pltpu.sync_copy(zf, out_hbm.at[pl.ds(start + q * CHUNK,
                                                     CHUNK)])


@functools.partial(
    pl.kernel,
    out_type=jax.ShapeDtypeStruct((NC, NPAD, H), _f32),
    mesh=_mesh,
    scratch_types=[
        pltpu.VMEM((ECH_T, EC), jnp.int32),
        pltpu.VMEM((EC,), jnp.int32),
        pltpu.VMEM((EC,), jnp.int32),
        pltpu.VMEM((CHUNK, H), _f32),
        pltpu.VMEM((CHUNK, H), _f32),
        pltpu.VMEM_SHARED((NPAD, H), _f32),
        pltpu.SemaphoreType.DMA,
        pltpu.SemaphoreType.DMA,
        pltpu.SemaphoreType.DMA,
        pltpu.SemaphoreType.DMA,
        pltpu.SemaphoreType.DMA,
        pltpu.SemaphoreType.DMA,
    ],
)
def _sc_prop(y_hbm, src_hbm, dst_hbm, out_hbm, sidx, dia, dib, bufa, bufb,
             acc, sema, semb, dsema, dsemb, ssema, ssemb):
    c = lax.axis_index("c")
    s = lax.axis_index("s")
    start = s * ROWS_T
    wid = s * NC + c
    base = wid * ECH_T

    pltpu.sync_copy(src_hbm.at[pl.ds(base, ECH_T)], sidx)

    def zrow(r, carry):
        for k in range(H // 16):
            bufa[r, pl.ds(k * 16, 16)] = jnp.zeros((16,), _f32)
        return carry

    lax.fori_loop(0, CHUNK, zrow, 0)
    zds = [pltpu.async_copy(bufa, acc.at[pl.ds(start + q * CHUNK, CHUNK)],
                            dsema) for q in range(ROWS_T // CHUNK)]
    pltpu.sync_copy(dst_hbm.at[base], dia)
    prime = pltpu.async_copy(y_hbm.at[sidx.at[0]], bufb.at[pl.ds(0, EC)],
                             semb)
    for zd in zds:
        zd.wait()
    plsc.subcore_barrier()
    prime.wait()

    def pair(j, carry):
        c0 = 2 * j
        ddb = pltpu.async_copy(dst_hbm.at[base + c0 + 1], dib, dsemb)
        sb = pltpu.async_copy(bufb.at[pl.ds(0, EC)], acc.at[dia], ssema,
                              add=True)
        da = pltpu.async_copy(y_hbm.at[sidx.at[c0 + 1]],
                              bufa.at[pl.ds(0, EC)], sema)
        ddb.wait()
        da.wait()
        sa = pltpu.async_copy(bufa.at[pl.ds(0, EC)], acc.at[dib], ssemb,
                              add=True)
        sb.wait()

        @pl.when(j < ECH_T // 2 - 1)
        def _():
            dda = pltpu.async_copy(dst_hbm.at[base + c0 + 2], dia, dsema)
            db = pltpu.async_copy(y_hbm.at[sidx.at[c0 + 2]],
                                  bufb.at[pl.ds(0, EC)], semb)
            dda.wait()
            db.wait()

        sa.wait()
        return carry

    lax.fori_loop(0, ECH_T // 2, pair, 0)
    plsc.subcore_barrier()
    pltpu.sync_copy(acc.at[pl.ds(start, ROWS_T)],
                    out_hbm.at[c, pl.ds(start, ROWS_T)])


def _pre_body(d0_ref, d1_ref, z_ref, dv_ref, y0_ref):
    deg = d0_ref[...] + d1_ref[...] + 1.0
    dv = jnp.transpose(lax.rsqrt(jnp.maximum(deg, 1.0)))
    dv_ref[...] = dv
    y0_ref[...] = z_ref[...] * dv[:N]


def _tc_pre(deg0, deg1, pearl_z):
    return pl.pallas_call(
        _pre_body,
        out_shape=[jax.ShapeDtypeStruct((NPAD, 1), _f32),
                   jax.ShapeDtypeStruct((N, H), _f32)],
    )(deg0.reshape(1, NPAD), deg1.reshape(1, NPAD), pearl_z)


def _dot(a, b):
    return jnp.dot(a, b, preferred_element_type=_f32)


def _s1_body(p_ref, y_ref, dv_ref, w_ref, b_ref, o_ref):
    dv = dv_ref[...]
    p = p_ref[...]
    u = (p[0] + p[1] + y_ref[...]) * dv
    z = jnp.maximum(_dot(u, w_ref[...]) + b_ref[...], 0.0)
    o_ref[...] = z * dv


def _tc_stage1(p, y, dinv_col, w, b):
    return pl.pallas_call(
        _s1_body,
        grid=(G,),
        in_specs=[pl.BlockSpec((NC, R, H), lambda i: (0, i, 0)),
                  pl.BlockSpec((R, H), lambda i: (i, 0)),
                  pl.BlockSpec((R, 1), lambda i: (i, 0)),
                  pl.BlockSpec((H, H), lambda i: (0, 0)),
                  pl.BlockSpec((1, H), lambda i: (0, 0))],
        out_specs=pl.BlockSpec((R, H), lambda i: (i, 0)),
        out_shape=jax.ShapeDtypeStruct((N, H), _f32),
    )(p, y, dinv_col, w, b)


def _s2_body(p_ref, y_ref, dv_ref, x_ref, wp2_ref, bp2_ref, wpo_ref, bpo_ref,
             wina_ref, winb_ref, bin_ref, wc1_ref, o_ref):
    dv = dv_ref[...]
    p = p_ref[...]
    u = (p[0] + p[1] + y_ref[...]) * dv
    z2 = jnp.maximum(_dot(u, wp2_ref[...]) + bp2_ref[...], 0.0)
    pe = _dot(z2, wpo_ref[...]) + bpo_ref[...]
    h0 = jnp.maximum(_dot(x_ref[...], wina_ref[...]) + _dot(pe, winb_ref[...])
                     + bin_ref[...], 0.0)
    t1 = _dot(h0, wc1_ref[...])
    o_ref[...] = t1 * dv


def _tc_stage2(p, y, dinv_col, x, wp2, bp2, wpo, bpo, wina, winb, b_in, wc1):
    return pl.pallas_call(
        _s2_body,
        grid=(G,),
        in_specs=[pl.BlockSpec((NC, R, H), lambda i: (0, i, 0)),
                  pl.BlockSpec((R, H), lambda i: (i, 0)),
                  pl.BlockSpec((R, 1), lambda i: (i, 0)),
                  pl.BlockSpec((R, D), lambda i: (i, 0)),
                  pl.BlockSpec((H, H), lambda i: (0, 0)),
                  pl.BlockSpec((1, H), lambda i: (0, 0)),
                  pl.BlockSpec((H, PE), lambda i: (0, 0)),
                  pl.BlockSpec((1, PE), lambda i: (0, 0)),
                  pl.BlockSpec((D, H), lambda i: (0, 0)),
                  pl.BlockSpec((PE, H), lambda i: (0, 0)),
                  pl.BlockSpec((1, H), lambda i: (0, 0)),
                  pl.BlockSpec((H, H), lambda i: (0, 0))],
        out_specs=pl.BlockSpec((R, H), lambda i: (i, 0)),
        out_shape=jax.ShapeDtypeStruct((N, H), _f32),
    )(p, y, dinv_col, x, wp2, bp2, wpo, bpo, wina, winb, b_in, wc1)


def _mid_body(residual, p_ref, y_ref, dv_ref, bc_ref, g_ref, be_ref, w_ref,
              *rest):
    if residual:
        hprev_ref, oy_ref, oh_ref = rest
    else:
        oy_ref, oh_ref = rest
    dv = dv_ref[...]
    p = p_ref[...]
    u = (p[0] + p[1] + y_ref[...]) * dv
    h = jnp.maximum((u + bc_ref[...]) * (g_ref[...] * BN_SCALE) + be_ref[...],
                    0.0)
    if residual:
        h = h + hprev_ref[...]
    oh_ref[...] = h
    oy_ref[...] = _dot(h, w_ref[...]) * dv


def _tc_mid(p, y, dinv_col, bc, g, be, w, hprev=None):
    residual = hprev is not None
    specs = [pl.BlockSpec((NC, R, H), lambda i: (0, i, 0)),
             pl.BlockSpec((R, H), lambda i: (i, 0)),
             pl.BlockSpec((R, 1), lambda i: (i, 0)),
             pl.BlockSpec((1, H), lambda i: (0, 0)),
             pl.BlockSpec((1, H), lambda i: (0, 0)),
             pl.BlockSpec((1, H), lambda i: (0, 0)),
             pl.BlockSpec((H, H), lambda i: (0, 0))]
    args = [p, y, dinv_col, bc, g, be, w]
    if residual:
        specs.append(pl.BlockSpec((R, H), lambda i: (i, 0)))
        args.append(hprev)
    return pl.pallas_call(
        functools.partial(_mid_body, residual),
        grid=(G,),
        in_specs=specs,
        out_specs=[pl.BlockSpec((R, H), lambda i: (i, 0)),
                   pl.BlockSpec((R, H), lambda i: (i, 0))],
        out_shape=[jax.ShapeDtypeStruct((N, H), _f32),
                   jax.ShapeDtypeStruct((N, H), _f32)],
    )(*args)


def _s5_body(p_ref, y_ref, dv_ref, bc_ref, g_ref, be_ref, hprev_ref,
             wout_ref, bout_ref, o_ref, acc_ref):
    i = pl.program_id(0)
    dv = dv_ref[...]
    p = p_ref[...]
    u = (p[0] + p[1] + y_ref[...]) * dv
    h = jnp.maximum((u + bc_ref[...]) * (g_ref[...] * BN_SCALE) + be_ref[...],
                    0.0) + hprev_ref[...]
    part = jnp.sum(h, axis=0, keepdims=True)

    @pl.when(i == 0)
    def _():
        acc_ref[...] = part

    @pl.when(i > 0)
    def _():
        acc_ref[...] = acc_ref[...] + part

    @pl.when(i == G - 1)
    def _():
        o_ref[...] = _dot(acc_ref[...] * (1.0 / N), wout_ref[...]) \
            + bout_ref[...]


def _tc_stage5(p, y, dinv_col, bc, g, be, hprev, wout, bout):
    return pl.pallas_call(
        _s5_body,
        grid=(G,),
        in_specs=[pl.BlockSpec((NC, R, H), lambda i: (0, i, 0)),
                  pl.BlockSpec((R, H), lambda i: (i, 0)),
                  pl.BlockSpec((R, 1), lambda i: (i, 0)),
                  pl.BlockSpec((1, H), lambda i: (0, 0)),
                  pl.BlockSpec((1, H), lambda i: (0, 0)),
                  pl.BlockSpec((1, H), lambda i: (0, 0)),
                  pl.BlockSpec((R, H), lambda i: (i, 0)),
                  pl.BlockSpec((H, OUT), lambda i: (0, 0)),
                  pl.BlockSpec((1, OUT), lambda i: (0, 0))],
        out_specs=pl.BlockSpec((1, OUT), lambda i: (0, 0)),
        out_shape=jax.ShapeDtypeStruct((1, OUT), _f32),
        scratch_shapes=[pltpu.VMEM((1, H), _f32)],
    )(p, y, dinv_col, bc, g, be, hprev, wout, bout)


def kernel(x, edge_index, pearl_z, Wp1, bp1, Wp2, bp2, Wpo, bpo, Win, b_in,
           Wc1, bc1, Wc2, bc2, Wc3, bc3, g1, be1, g2, be2, g3, be3,
           Wout, bout):
    src = edge_index[0]
    dst = edge_index[1]
    src2d = src.reshape(E // EC, EC)
    dst2d = dst.reshape(E // EC, EC)
    row = lambda v: v.reshape(1, -1)

    deg0, deg1 = _sc_degree(dst2d)
    dinv_col, y0 = _tc_pre(deg0, deg1, pearl_z)
    p = _sc_prop(y0, src2d, dst2d)
    y1 = _tc_stage1(p, y0, dinv_col, Wp1, row(bp1))
    p = _sc_prop(y1, src2d, dst2d)
    y2 = _tc_stage2(p, y1, dinv_col, x, Wp2, row(bp2), Wpo, row(bpo),
                    Win[:D], Win[D:], row(b_in), Wc1)
    p = _sc_prop(y2, src2d, dst2d)
    y3, h1 = _tc_mid(p, y2, dinv_col, row(bc1), row(g1), row(be1), Wc2)
    p = _sc_prop(y3, src2d, dst2d)
    y4, h2 = _tc_mid(p, y3, dinv_col, row(bc2), row(g2), row(be2), Wc3,
                     hprev=h1)
    p = _sc_prop(y4, src2d, dst2d)
    return _tc_stage5(p, y4, dinv_col, row(bc3), row(g3), row(be3), h2,
                      Wout, row(bout))
